# baseline (speedup 1.0000x reference)
import functools

import jax
import jax.numpy as jnp
from jax import lax
from jax.experimental import pallas as pl
from jax.experimental.pallas import tpu as pltpu
from jax.experimental.pallas import tpu_sc as plsc

N = 10000
E = 320000
D = 128

NC = 2
NS = 16
NW = NC * NS
EW = E // NW
K = 50
CH = EW // K
IB = 20
NSLOT = 5
NBLK = CH // IB
NP = 10240
RPT = NP // NS
RZC = 40
KF = K // 16
KR = K - 16 * KF


def _make_sc_segsum(with_deg):
  mesh = plsc.VectorSubcoreMesh(core_axis_name="c", subcore_axis_name="s")

  out_type = jax.ShapeDtypeStruct((2 * NP, D), jnp.float32)
  scratch = [
      pltpu.VMEM((IB, K), jnp.int32),
      pltpu.VMEM((IB, K), jnp.int32),
      pltpu.VMEM_SHARED((NP, D), jnp.float32),
  ]
  scratch += [pltpu.VMEM((K, D), jnp.float32) for _ in range(NSLOT)]
  scratch += [pltpu.SemaphoreType.DMA for _ in range(2 * NSLOT)]
  if with_deg:
    out_type = [out_type, jax.ShapeDtypeStruct((NW, NP), jnp.float32)]
    scratch = scratch + [
        pltpu.VMEM((NP,), jnp.float32),
    ]

  @functools.partial(
      pl.kernel,
      out_type=out_type,
      mesh=mesh,
      compiler_params=pltpu.CompilerParams(use_tc_tiling_on_sc=False,
                                           needs_layout_passes=False),
      scratch_types=scratch,
  )
  def segsum(tab_hbm, edges_hbm, out_hbm, *rest):
    if with_deg:
      deg_hbm = rest[0]
      rest = rest[1:]
    src_v, dst_v, acc_sh = rest[0], rest[1], rest[2]
    rows_bufs = rest[3:3 + NSLOT]
    sems = rest[3 + NSLOT:3 + 3 * NSLOT]
    if with_deg:
      degloc = rest[3 + 3 * NSLOT]
    rows0 = rows_bufs[0]
    slots = tuple((rows_bufs[t], sems[2 * t], sems[2 * t + 1])
                  for t in range(NSLOT))
    cid = lax.axis_index("c")
    sid = lax.axis_index("s")
    wid = cid * NS + sid

    zero = jnp.zeros((16,), jnp.float32)

    def zrow(r, carry):
      def zcol(c, carry2):
        rows0[r, pl.ds(c * 16, 16)] = zero
        return carry2
      return lax.fori_loop(0, D // 16, zcol, carry)

    lax.fori_loop(0, RZC, zrow, 0)
    for j in range(RPT // RZC):
      pltpu.sync_copy(rows0.at[pl.ds(0, RZC)],
                      acc_sh.at[pl.ds(sid * RPT + j * RZC, RZC)])

    if with_deg:
      def zdeg(r, carry):
        degloc[pl.ds(r * 16, 16)] = zero
        return carry
      lax.fori_loop(0, NP // 16, zdeg, 0)

    pltpu.sync_copy(edges_hbm.at[0, wid, pl.ds(0, IB)], src_v)
    pltpu.sync_copy(edges_hbm.at[1, wid, pl.ds(0, IB)], dst_v)

    plsc.subcore_barrier()

    def g_issue(rows, c, sem):
      pltpu.async_copy(tab_hbm.at[src_v.at[c]], rows, sem)

    def g_wait(rows, c, sem):
      pltpu.make_async_copy(tab_hbm.at[src_v.at[c]], rows, sem).wait()

    def s_issue(rows, c, sem):
      pltpu.async_copy(rows, acc_sh.at[dst_v.at[c]], sem, add=True)

    def s_wait(rows, c, sem):
      pltpu.make_async_copy(rows, acc_sh.at[dst_v.at[c]], sem).wait()

    if with_deg:
      ones16 = jnp.ones((16,), jnp.float32)
      rmask = lax.iota(jnp.int32, 16) >= 16 - KR

      def histo(c):
        for g in range(KF):
          idx = dst_v[c, pl.ds(g * 16, 16)]
          plsc.addupdate_scatter(degloc, [idx], ones16)
        if KR:
          idx = dst_v[c, pl.ds(K - 16, 16)]
          plsc.addupdate_scatter(degloc, [idx], ones16, mask=rmask)
    else:
      def histo(c):
        del c

    for b in range(NBLK):
      if b > 0:
        pltpu.sync_copy(edges_hbm.at[0, wid, pl.ds(b * IB, IB)], src_v)
        pltpu.sync_copy(edges_hbm.at[1, wid, pl.ds(b * IB, IB)], dst_v)

      for t, (rows, sg, ss) in enumerate(slots):
        g_issue(rows, t, sg)

      def group(q, carry):
        c0 = NSLOT * q
        for t, (rows, sg, ss) in enumerate(slots):
          g_wait(rows, c0 + t, sg)
          s_issue(rows, c0 + t, ss)
          histo(c0 + t)
        for t, (rows, sg, ss) in enumerate(slots):
          s_wait(rows, c0 + t, ss)
          g_issue(rows, c0 + t + NSLOT, sg)
        return carry

      lax.fori_loop(0, IB // NSLOT - 1, group, 0)

      c0 = IB - NSLOT
      for t, (rows, sg, ss) in enumerate(slots):
        g_wait(rows, c0 + t, sg)
        s_issue(rows, c0 + t, ss)
        histo(c0 + t)
      for t, (rows, sg, ss) in enumerate(slots):
        s_wait(rows, c0 + t, ss)

    if with_deg:
      pltpu.sync_copy(degloc, deg_hbm.at[wid])

    plsc.subcore_barrier()

    row = sid * RPT
    pltpu.sync_copy(acc_sh.at[pl.ds(row, RPT)],
                    out_hbm.at[pl.ds(cid * NP + row, RPT)])

  return segsum


_sc_segsum_deg = _make_sc_segsum(True)
_sc_segsum = _make_sc_segsum(False)


_RB = 2048


def _dense_body(p_ref, d_ref, x_ref, wm_ref, ws_ref, b_ref, o_ref):
  a = p_ref[0] + p_ref[1]
  deg = jnp.sum(d_ref[...], axis=1, keepdims=True)
  scale = 1.0 / jnp.maximum(deg, 1.0)
  agg = jnp.dot(a * scale, wm_ref[...], preferred_element_type=jnp.float32)
  self_t = jnp.dot(x_ref[...], ws_ref[...], preferred_element_type=jnp.float32)
  o_ref[...] = jnp.maximum(agg + self_t + b_ref[...], 0.0)


def _dense_layer(P, degT, x, Wm, Ws, b):
  return pl.pallas_call(
      _dense_body,
      grid=(NP // _RB,),
      in_specs=[
          pl.BlockSpec((2, _RB, D), lambda i: (0, i, 0)),
          pl.BlockSpec((_RB, NW), lambda i: (i, 0)),
          pl.BlockSpec((_RB, D), lambda i: (i, 0)),
          pl.BlockSpec((D, D), lambda i: (0, 0)),
          pl.BlockSpec((D, D), lambda i: (0, 0)),
          pl.BlockSpec((1, D), lambda i: (0, 0)),
      ],
      out_specs=pl.BlockSpec((_RB, D), lambda i: (i, 0)),
      out_shape=jax.ShapeDtypeStruct((NP, D), jnp.float32),
  )(P, degT, x, Wm, Ws, b.reshape(1, D))


def _dense2_head_body(p_ref, d_ref, x_ref, wm_ref, ws_ref, b_ref,
                      w1_ref, b1_ref, w2_ref, b2_ref, o_ref, gmax):
  i = pl.program_id(0)
  a = p_ref[0] + p_ref[1]
  deg = jnp.sum(d_ref[...], axis=1, keepdims=True)
  scale = 1.0 / jnp.maximum(deg, 1.0)
  agg = jnp.dot(a * scale, wm_ref[...], preferred_element_type=jnp.float32)
  self_t = jnp.dot(x_ref[...], ws_ref[...], preferred_element_type=jnp.float32)
  h2 = jnp.maximum(agg + self_t + b_ref[...], 0.0)
  rows = i * _RB + lax.broadcasted_iota(jnp.int32, (_RB, 1), 0)
  bm = jnp.max(jnp.where(rows < N, h2, -jnp.inf), axis=0, keepdims=True)

  @pl.when(i == 0)
  def _():
    gmax[...] = bm

  @pl.when(i > 0)
  def _():
    gmax[...] = jnp.maximum(gmax[...], bm)

  @pl.when(i == NP // _RB - 1)
  def _():
    g = gmax[...]
    h1v = jnp.maximum(
        jnp.dot(g, w1_ref[...], preferred_element_type=jnp.float32)
        + b1_ref[...], 0.0)
    o_ref[...] = jnp.dot(h1v, w2_ref[...],
                         preferred_element_type=jnp.float32) + b2_ref[...]


def _dense2_head(P, degT, x, Wm, Ws, b, fc1_w, fc1_b, fc2_w, fc2_b):
  return pl.pallas_call(
      _dense2_head_body,
      grid=(NP // _RB,),
      in_specs=[
          pl.BlockSpec((2, _RB, D), lambda i: (0, i, 0)),
          pl.BlockSpec((_RB, NW), lambda i: (i, 0)),
          pl.BlockSpec((_RB, D), lambda i: (i, 0)),
          pl.BlockSpec((D, D), lambda i: (0, 0)),
          pl.BlockSpec((D, D), lambda i: (0, 0)),
          pl.BlockSpec((1, D), lambda i: (0, 0)),
          pl.BlockSpec((D, D // 2), lambda i: (0, 0)),
          pl.BlockSpec((1, D // 2), lambda i: (0, 0)),
          pl.BlockSpec((D // 2, D), lambda i: (0, 0)),
          pl.BlockSpec((1, D), lambda i: (0, 0)),
      ],
      out_specs=pl.BlockSpec((1, D), lambda i: (0, 0)),
      out_shape=jax.ShapeDtypeStruct((1, D), jnp.float32),
      scratch_shapes=[pltpu.VMEM((1, D), jnp.float32)],
  )(P, degT, x, Wm, Ws, b.reshape(1, D),
    fc1_w, fc1_b.reshape(1, -1), fc2_w, fc2_b.reshape(1, D))


def kernel(x, edge_index, W_msg0, W_self0, b0, W_msg1, W_self1, b1,
           fc1_w, fc1_b, fc2_w, fc2_b):
  edges = edge_index.astype(jnp.int32).reshape(2, NW, CH, K)

  x_pad = jnp.pad(x, ((0, NP - N), (0, 0)))

  P1, deg = _sc_segsum_deg(x_pad, edges)
  degT = jnp.transpose(deg)
  h1 = _dense_layer(P1.reshape(2, NP, D), degT, x_pad, W_msg0, W_self0, b0)

  P2 = _sc_segsum(h1, edges)
  return _dense2_head(P2.reshape(2, NP, D), degT, h1, W_msg1, W_self1, b1,
                      fc1_w, fc1_b, fc2_w, fc2_b)


# speedup vs baseline: 1.0924x; 1.0924x over previous
import functools

import jax
import jax.numpy as jnp
from jax import lax
from jax.experimental import pallas as pl
from jax.experimental.pallas import tpu as pltpu
from jax.experimental.pallas import tpu_sc as plsc

N = 10000
E = 320000
D = 128

NC = 2
NS = 16
NW = NC * NS
EW = E // NW
K = 50
CH = EW // K
IB = 100
NSLOT = 4
NBLK = CH // IB
NP = 10240
RPT = NP // NS
RZC = 40
KF = K // 16
KR = K - 16 * KF


def _make_sc_segsum(with_deg):
  mesh = plsc.VectorSubcoreMesh(core_axis_name="c", subcore_axis_name="s")

  out_type = jax.ShapeDtypeStruct((2 * NP, D), jnp.float32)
  scratch = [
      pltpu.VMEM((IB, K), jnp.int32),
      pltpu.VMEM((IB, K), jnp.int32),
      pltpu.VMEM_SHARED((NP, D), jnp.float32),
  ]
  scratch += [pltpu.VMEM((K, D), jnp.float32) for _ in range(NSLOT)]
  scratch += [pltpu.SemaphoreType.DMA for _ in range(2 * NSLOT)]
  if with_deg:
    out_type = [out_type, jax.ShapeDtypeStruct((NW, NP), jnp.float32)]
    scratch = scratch + [
        pltpu.VMEM((NP,), jnp.float32),
    ]

  @functools.partial(
      pl.kernel,
      out_type=out_type,
      mesh=mesh,
      compiler_params=pltpu.CompilerParams(use_tc_tiling_on_sc=False,
                                           needs_layout_passes=False),
      scratch_types=scratch,
  )
  def segsum(tab_hbm, edges_hbm, out_hbm, *rest):
    if with_deg:
      deg_hbm = rest[0]
      rest = rest[1:]
    src_v, dst_v, acc_sh = rest[0], rest[1], rest[2]
    rows_bufs = rest[3:3 + NSLOT]
    sems = rest[3 + NSLOT:3 + 3 * NSLOT]
    if with_deg:
      degloc = rest[3 + 3 * NSLOT]
    rows0 = rows_bufs[0]
    slots = tuple((rows_bufs[t], sems[2 * t], sems[2 * t + 1])
                  for t in range(NSLOT))
    cid = lax.axis_index("c")
    sid = lax.axis_index("s")
    wid = cid * NS + sid

    zero = jnp.zeros((16,), jnp.float32)

    def zrow(r, carry):
      def zcol(c, carry2):
        rows0[r, pl.ds(c * 16, 16)] = zero
        return carry2
      return lax.fori_loop(0, D // 16, zcol, carry)

    lax.fori_loop(0, RZC, zrow, 0)
    for j in range(RPT // RZC):
      pltpu.sync_copy(rows0.at[pl.ds(0, RZC)],
                      acc_sh.at[pl.ds(sid * RPT + j * RZC, RZC)])

    if with_deg:
      def zdeg(r, carry):
        degloc[pl.ds(r * 16, 16)] = zero
        return carry
      lax.fori_loop(0, NP // 16, zdeg, 0)

    pltpu.sync_copy(edges_hbm.at[0, wid, pl.ds(0, IB)], src_v)
    pltpu.sync_copy(edges_hbm.at[1, wid, pl.ds(0, IB)], dst_v)

    plsc.subcore_barrier()

    def g_issue(rows, c, sem):
      pltpu.async_copy(tab_hbm.at[src_v.at[c]], rows, sem)

    def g_wait(rows, c, sem):
      pltpu.make_async_copy(tab_hbm.at[src_v.at[c]], rows, sem).wait()

    def s_issue(rows, c, sem):
      pltpu.async_copy(rows, acc_sh.at[dst_v.at[c]], sem, add=True)

    def s_wait(rows, c, sem):
      pltpu.make_async_copy(rows, acc_sh.at[dst_v.at[c]], sem).wait()

    if with_deg:
      ones16 = jnp.ones((16,), jnp.float32)
      rmask = lax.iota(jnp.int32, 16) >= 16 - KR

      def histo(c):
        for g in range(KF):
          idx = dst_v[c, pl.ds(g * 16, 16)]
          plsc.addupdate_scatter(degloc, [idx], ones16)
        if KR:
          idx = dst_v[c, pl.ds(K - 16, 16)]
          plsc.addupdate_scatter(degloc, [idx], ones16, mask=rmask)
    else:
      def histo(c):
        del c

    for b in range(NBLK):
      if b > 0:
        pltpu.sync_copy(edges_hbm.at[0, wid, pl.ds(b * IB, IB)], src_v)
        pltpu.sync_copy(edges_hbm.at[1, wid, pl.ds(b * IB, IB)], dst_v)

      for t, (rows, sg, ss) in enumerate(slots):
        g_issue(rows, t, sg)

      def group(q, carry):
        c0 = NSLOT * q
        for t, (rows, sg, ss) in enumerate(slots):
          g_wait(rows, c0 + t, sg)
          s_issue(rows, c0 + t, ss)
          histo(c0 + t)
        for t, (rows, sg, ss) in enumerate(slots):
          s_wait(rows, c0 + t, ss)
          g_issue(rows, c0 + t + NSLOT, sg)
        return carry

      lax.fori_loop(0, IB // NSLOT - 1, group, 0)

      c0 = IB - NSLOT
      for t, (rows, sg, ss) in enumerate(slots):
        g_wait(rows, c0 + t, sg)
        s_issue(rows, c0 + t, ss)
        histo(c0 + t)
      for t, (rows, sg, ss) in enumerate(slots):
        s_wait(rows, c0 + t, ss)

    if with_deg:
      pltpu.sync_copy(degloc, deg_hbm.at[wid])

    plsc.subcore_barrier()

    row = sid * RPT
    pltpu.sync_copy(acc_sh.at[pl.ds(row, RPT)],
                    out_hbm.at[pl.ds(cid * NP + row, RPT)])

  return segsum


_sc_segsum_deg = _make_sc_segsum(True)
_sc_segsum = _make_sc_segsum(False)


_RB = 2048


def _dense_body(p_ref, d_ref, x_ref, wm_ref, ws_ref, b_ref, o_ref):
  a = p_ref[0] + p_ref[1]
  deg = jnp.sum(d_ref[...], axis=1, keepdims=True)
  scale = 1.0 / jnp.maximum(deg, 1.0)
  agg = jnp.dot(a * scale, wm_ref[...], preferred_element_type=jnp.float32)
  self_t = jnp.dot(x_ref[...], ws_ref[...], preferred_element_type=jnp.float32)
  o_ref[...] = jnp.maximum(agg + self_t + b_ref[...], 0.0)


def _dense_layer(P, degT, x, Wm, Ws, b):
  return pl.pallas_call(
      _dense_body,
      grid=(NP // _RB,),
      in_specs=[
          pl.BlockSpec((2, _RB, D), lambda i: (0, i, 0)),
          pl.BlockSpec((_RB, NW), lambda i: (i, 0)),
          pl.BlockSpec((_RB, D), lambda i: (i, 0)),
          pl.BlockSpec((D, D), lambda i: (0, 0)),
          pl.BlockSpec((D, D), lambda i: (0, 0)),
          pl.BlockSpec((1, D), lambda i: (0, 0)),
      ],
      out_specs=pl.BlockSpec((_RB, D), lambda i: (i, 0)),
      out_shape=jax.ShapeDtypeStruct((NP, D), jnp.float32),
  )(P, degT, x, Wm, Ws, b.reshape(1, D))


def _dense2_head_body(p_ref, d_ref, x_ref, wm_ref, ws_ref, b_ref,
                      w1_ref, b1_ref, w2_ref, b2_ref, o_ref, gmax):
  i = pl.program_id(0)
  a = p_ref[0] + p_ref[1]
  deg = jnp.sum(d_ref[...], axis=1, keepdims=True)
  scale = 1.0 / jnp.maximum(deg, 1.0)
  agg = jnp.dot(a * scale, wm_ref[...], preferred_element_type=jnp.float32)
  self_t = jnp.dot(x_ref[...], ws_ref[...], preferred_element_type=jnp.float32)
  h2 = jnp.maximum(agg + self_t + b_ref[...], 0.0)
  rows = i * _RB + lax.broadcasted_iota(jnp.int32, (_RB, 1), 0)
  bm = jnp.max(jnp.where(rows < N, h2, -jnp.inf), axis=0, keepdims=True)

  @pl.when(i == 0)
  def _():
    gmax[...] = bm

  @pl.when(i > 0)
  def _():
    gmax[...] = jnp.maximum(gmax[...], bm)

  @pl.when(i == NP // _RB - 1)
  def _():
    g = gmax[...]
    h1v = jnp.maximum(
        jnp.dot(g, w1_ref[...], preferred_element_type=jnp.float32)
        + b1_ref[...], 0.0)
    o_ref[...] = jnp.dot(h1v, w2_ref[...],
                         preferred_element_type=jnp.float32) + b2_ref[...]


def _dense2_head(P, degT, x, Wm, Ws, b, fc1_w, fc1_b, fc2_w, fc2_b):
  return pl.pallas_call(
      _dense2_head_body,
      grid=(NP // _RB,),
      in_specs=[
          pl.BlockSpec((2, _RB, D), lambda i: (0, i, 0)),
          pl.BlockSpec((_RB, NW), lambda i: (i, 0)),
          pl.BlockSpec((_RB, D), lambda i: (i, 0)),
          pl.BlockSpec((D, D), lambda i: (0, 0)),
          pl.BlockSpec((D, D), lambda i: (0, 0)),
          pl.BlockSpec((1, D), lambda i: (0, 0)),
          pl.BlockSpec((D, D // 2), lambda i: (0, 0)),
          pl.BlockSpec((1, D // 2), lambda i: (0, 0)),
          pl.BlockSpec((D // 2, D), lambda i: (0, 0)),
          pl.BlockSpec((1, D), lambda i: (0, 0)),
      ],
      out_specs=pl.BlockSpec((1, D), lambda i: (0, 0)),
      out_shape=jax.ShapeDtypeStruct((1, D), jnp.float32),
      scratch_shapes=[pltpu.VMEM((1, D), jnp.float32)],
  )(P, degT, x, Wm, Ws, b.reshape(1, D),
    fc1_w, fc1_b.reshape(1, -1), fc2_w, fc2_b.reshape(1, D))


def kernel(x, edge_index, W_msg0, W_self0, b0, W_msg1, W_self1, b1,
           fc1_w, fc1_b, fc2_w, fc2_b):
  edges = edge_index.astype(jnp.int32).reshape(2, NW, CH, K)

  x_pad = jnp.pad(x, ((0, NP - N), (0, 0)))

  P1, deg = _sc_segsum_deg(x_pad, edges)
  degT = jnp.transpose(deg)
  h1 = _dense_layer(P1.reshape(2, NP, D), degT, x_pad, W_msg0, W_self0, b0)

  P2 = _sc_segsum(h1, edges)
  return _dense2_head(P2.reshape(2, NP, D), degT, h1, W_msg1, W_self1, b1,
                      fc1_w, fc1_b, fc2_w, fc2_b)
